# baseline (speedup 1.0000x reference)
import functools

import jax
import jax.numpy as jnp
from jax import lax
from jax.experimental import pallas as pl
from jax.experimental.pallas import tpu as pltpu
from jax.experimental.pallas import tpu_sc as plsc

N_USERS = 1000000
N_ITEMS = 1000000
EMB = 32
BATCH = 16384

_INFO = plsc.get_sparse_core_info()
_NC = _INFO.num_cores
_NS = _INFO.num_subcores
_L = _INFO.num_lanes
_NW = _NC * _NS
_BPW = BATCH // _NW
_IDX_CHUNK = 128
_NXFER = _BPW * EMB // _IDX_CHUNK


def _mf_kernel(u_id_hbm, i_id_hbm, uf_hbm, if_hbm, out_hbm,
               uid_v, iid_v, uidx_v, iidx_v, urows_v, irows_v, out_v, sem):
    wid = lax.axis_index("s") * _NC + lax.axis_index("c")
    base = wid * _BPW

    pltpu.sync_copy(u_id_hbm.at[pl.ds(base, _BPW)], uid_v)
    pltpu.sync_copy(i_id_hbm.at[pl.ds(base, _BPW)], iid_v)

    def gen(g, carry):
        uid = uid_v[pl.ds(g * _L, _L)]
        iid = iid_v[pl.ds(g * _L, _L)]
        for e in range(EMB):
            uidx_v[pl.ds(e * _BPW + g * _L, _L)] = uid + e * N_USERS
            iidx_v[pl.ds(e * _BPW + g * _L, _L)] = iid + e * N_ITEMS
        return carry

    lax.fori_loop(0, _BPW // _L, gen, 0, unroll=False)

    copies = []
    for c in range(_NXFER):
        sl = pl.ds(c * _IDX_CHUNK, _IDX_CHUNK)
        copies.append(pltpu.async_copy(uf_hbm.at[uidx_v.at[sl]],
                                       urows_v.at[sl], sem))
        copies.append(pltpu.async_copy(if_hbm.at[iidx_v.at[sl]],
                                       irows_v.at[sl], sem))
    for cp in copies:
        cp.wait()

    def body(g, carry):
        acc = jnp.zeros((_L,), jnp.float32)
        for e in range(EMB):
            sl = pl.ds(e * _BPW + g * _L, _L)
            acc = acc + urows_v[sl] * irows_v[sl]
        out_v[pl.ds(g * _L, _L)] = acc
        return carry

    lax.fori_loop(0, _BPW // _L, body, 0, unroll=False)

    pltpu.sync_copy(out_v, out_hbm.at[pl.ds(base, _BPW)])


@functools.partial(jax.jit)
def kernel(u_id, i_id, user_factors, item_factors):
    u_id = u_id.astype(jnp.int32)
    i_id = i_id.astype(jnp.int32)
    uf_flat = user_factors.T.reshape(N_USERS * EMB)
    if_flat = item_factors.T.reshape(N_ITEMS * EMB)
    mesh = plsc.VectorSubcoreMesh(core_axis_name="c", subcore_axis_name="s")
    run = pl.kernel(
        _mf_kernel,
        mesh=mesh,
        out_type=jax.ShapeDtypeStruct((BATCH,), jnp.float32),
        scratch_types=[
            pltpu.VMEM((_BPW,), jnp.int32),
            pltpu.VMEM((_BPW,), jnp.int32),
            pltpu.VMEM((_BPW * EMB,), jnp.int32),
            pltpu.VMEM((_BPW * EMB,), jnp.int32),
            pltpu.VMEM((_BPW * EMB,), jnp.float32),
            pltpu.VMEM((_BPW * EMB,), jnp.float32),
            pltpu.VMEM((_BPW,), jnp.float32),
            pltpu.SemaphoreType.DMA,
        ],
        compiler_params=pltpu.CompilerParams(
            needs_layout_passes=False, use_tc_tiling_on_sc=False),
    )
    return run(u_id, i_id, uf_flat, if_flat)


# speedup vs baseline: 8.5504x; 8.5504x over previous
import functools

import jax
import jax.numpy as jnp
from jax import lax
from jax.experimental import pallas as pl
from jax.experimental.pallas import tpu as pltpu
from jax.experimental.pallas import tpu_sc as plsc

N_USERS = 1000000
N_ITEMS = 1000000
EMB = 32
BATCH = 16384

_INFO = plsc.get_sparse_core_info()
_NC = _INFO.num_cores
_NS = _INFO.num_subcores
_L = _INFO.num_lanes
_NW = _NC * _NS
_BPW = BATCH // _NW
_IDX_CHUNK = 128
_NXFER = _BPW * EMB // _IDX_CHUNK

_LANES = 128
_TPC = N_USERS // _LANES
_TAIL = N_USERS - _TPC * _LANES
_G = 32
_NSTEP = 8
_NBAND = EMB // 8


def _detile_kernel(uT3, iT3, uflat, iflat, buf0, buf1, buf_t, rsem,
                   wsem0, wsem1):
    wid = lax.axis_index("s") * _NC + lax.axis_index("c")
    lo = (wid * (_TPC - _G * _NSTEP)) // (_NW - 1)

    @pl.when(wid == 0)
    def _tail():
        for src3, dstf, n in ((uT3, uflat, N_USERS), (iT3, iflat, N_ITEMS)):
            for c in range(_NBAND):
                for d in range(8):
                    pltpu.async_copy(
                        src3.at[c, d, pl.ds(_TPC * _LANES, _TAIL)],
                        buf_t, rsem).wait()
                    pltpu.async_copy(
                        buf_t,
                        dstf.at[pl.ds((c * 8 + d) * n + _TPC * _LANES,
                                      _TAIL)],
                        wsem0).wait()

    bufs = (buf0, buf1)
    wsems = (wsem0, wsem1)
    pending = [None, None]
    step = 0
    for src3, dstf, n in ((uT3, uflat, N_USERS), (iT3, iflat, N_ITEMS)):
        for c in range(_NBAND):
            for k in range(_NSTEP):
                col0 = (lo + k * _G) * _LANES
                for d in range(8):
                    s = step % 2
                    if pending[s] is not None:
                        pending[s].wait()
                    pltpu.async_copy(
                        src3.at[c, d, pl.ds(col0, _G * _LANES)],
                        bufs[s], rsem).wait()
                    pending[s] = pltpu.async_copy(
                        bufs[s],
                        dstf.at[pl.ds((c * 8 + d) * n + col0, _G * _LANES)],
                        wsems[s])
                    step += 1
    for cp in pending:
        if cp is not None:
            cp.wait()


def _mf_kernel(u_id_hbm, i_id_hbm, uf_hbm, if_hbm, out_hbm,
               uid_v, iid_v, uidx_v, iidx_v, urows_v, irows_v, out_v, sem):
    wid = lax.axis_index("s") * _NC + lax.axis_index("c")
    base = wid * _BPW

    pltpu.sync_copy(u_id_hbm.at[pl.ds(base, _BPW)], uid_v)
    pltpu.sync_copy(i_id_hbm.at[pl.ds(base, _BPW)], iid_v)

    def gen(g, carry):
        uid = uid_v[pl.ds(g * _L, _L)]
        iid = iid_v[pl.ds(g * _L, _L)]
        for e in range(EMB):
            uidx_v[pl.ds(e * _BPW + g * _L, _L)] = uid + e * N_USERS
            iidx_v[pl.ds(e * _BPW + g * _L, _L)] = iid + e * N_ITEMS
        return carry

    lax.fori_loop(0, _BPW // _L, gen, 0, unroll=False)

    copies = []
    for c in range(_NXFER):
        sl = pl.ds(c * _IDX_CHUNK, _IDX_CHUNK)
        copies.append(pltpu.async_copy(uf_hbm.at[uidx_v.at[sl]],
                                       urows_v.at[sl], sem))
        copies.append(pltpu.async_copy(if_hbm.at[iidx_v.at[sl]],
                                       irows_v.at[sl], sem))
    for cp in copies:
        cp.wait()

    def body(g, carry):
        acc = jnp.zeros((_L,), jnp.float32)
        for e in range(EMB):
            sl = pl.ds(e * _BPW + g * _L, _L)
            acc = acc + urows_v[sl] * irows_v[sl]
        out_v[pl.ds(g * _L, _L)] = acc
        return carry

    lax.fori_loop(0, _BPW // _L, body, 0, unroll=False)

    pltpu.sync_copy(out_v, out_hbm.at[pl.ds(base, _BPW)])


@functools.partial(jax.jit)
def kernel(u_id, i_id, user_factors, item_factors):
    u_id = u_id.astype(jnp.int32)
    i_id = i_id.astype(jnp.int32)
    mesh = plsc.VectorSubcoreMesh(core_axis_name="c", subcore_axis_name="s")

    uT3 = user_factors.T.reshape(_NBAND, 8, N_USERS)
    iT3 = item_factors.T.reshape(_NBAND, 8, N_ITEMS)

    detile = pl.kernel(
        _detile_kernel,
        mesh=mesh,
        out_type=(jax.ShapeDtypeStruct((N_USERS * EMB,), jnp.float32),
                  jax.ShapeDtypeStruct((N_ITEMS * EMB,), jnp.float32)),
        scratch_types=[
            pltpu.VMEM((_G * _LANES,), jnp.float32),
            pltpu.VMEM((_G * _LANES,), jnp.float32),
            pltpu.VMEM((_TAIL,), jnp.float32),
            pltpu.SemaphoreType.DMA,
            pltpu.SemaphoreType.DMA,
            pltpu.SemaphoreType.DMA,
        ],
        compiler_params=pltpu.CompilerParams(needs_layout_passes=False),
    )
    uf_flat, if_flat = detile(uT3, iT3)

    run = pl.kernel(
        _mf_kernel,
        mesh=mesh,
        out_type=jax.ShapeDtypeStruct((BATCH,), jnp.float32),
        scratch_types=[
            pltpu.VMEM((_BPW,), jnp.int32),
            pltpu.VMEM((_BPW,), jnp.int32),
            pltpu.VMEM((_BPW * EMB,), jnp.int32),
            pltpu.VMEM((_BPW * EMB,), jnp.int32),
            pltpu.VMEM((_BPW * EMB,), jnp.float32),
            pltpu.VMEM((_BPW * EMB,), jnp.float32),
            pltpu.VMEM((_BPW,), jnp.float32),
            pltpu.SemaphoreType.DMA,
        ],
        compiler_params=pltpu.CompilerParams(
            needs_layout_passes=False, use_tc_tiling_on_sc=False),
    )
    return run(u_id, i_id, uf_flat, if_flat)


# speedup vs baseline: 18.7203x; 2.1894x over previous
import functools

import jax
import jax.numpy as jnp
from jax import lax
from jax.experimental import pallas as pl
from jax.experimental.pallas import tpu as pltpu
from jax.experimental.pallas import tpu_sc as plsc

N_USERS = 1000000
N_ITEMS = 1000000
EMB = 32
BATCH = 16384

_INFO = plsc.get_sparse_core_info()
_NC = _INFO.num_cores
_NS = _INFO.num_subcores
_L = _INFO.num_lanes
_NW = _NC * _NS
_BPW = BATCH // _NW
_IDX_CHUNK = 128
_NXFER = _BPW * EMB // _IDX_CHUNK

_LANES = 128
_TPC = N_USERS // _LANES
_TAIL = N_USERS - _TPC * _LANES
_G = 64
_NSTEP = 4
_NBAND = EMB // 8
_NBUF = 8
_RDEPTH = 4


def _detile_kernel(uT3, iT3, uflat, iflat, *scratch):
    bufs = scratch[:_NBUF]
    buf_t = scratch[_NBUF]
    rsems = scratch[_NBUF + 1:2 * _NBUF + 1]
    wsems = scratch[2 * _NBUF + 1:]
    wid = lax.axis_index("s") * _NC + lax.axis_index("c")
    lo = (wid * (_TPC - _G * _NSTEP)) // (_NW - 1)

    @pl.when(wid == 0)
    def _tail():
        for src3, dstf, n in ((uT3, uflat, N_USERS), (iT3, iflat, N_ITEMS)):
            for c in range(_NBAND):
                for d in range(8):
                    pltpu.async_copy(
                        src3.at[c, d, pl.ds(_TPC * _LANES, _TAIL)],
                        buf_t, rsems[0]).wait()
                    pltpu.async_copy(
                        buf_t,
                        dstf.at[pl.ds((c * 8 + d) * n + _TPC * _LANES,
                                      _TAIL)],
                        wsems[0]).wait()

    steps = []
    for src3, dstf, n in ((uT3, uflat, N_USERS), (iT3, iflat, N_ITEMS)):
        for c in range(_NBAND):
            for k in range(_NSTEP):
                for d in range(8):
                    steps.append((src3, dstf, n, c, k, d))
    nsteps = len(steps)
    pending_r = [None] * _NBUF
    pending_w = [None] * _NBUF
    for t in range(nsteps + _RDEPTH):
        if t < nsteps:
            b = t % _NBUF
            src3, dstf, n, c, k, d = steps[t]
            if pending_w[b] is not None:
                pending_w[b].wait()
            col0 = (lo + k * _G) * _LANES
            pending_r[b] = pltpu.async_copy(
                src3.at[c, d, pl.ds(col0, _G * _LANES)], bufs[b], rsems[b])
        tw = t - _RDEPTH
        if tw >= 0:
            bw = tw % _NBUF
            src3, dstf, n, c, k, d = steps[tw]
            pending_r[bw].wait()
            col0 = (lo + k * _G) * _LANES
            pending_w[bw] = pltpu.async_copy(
                bufs[bw],
                dstf.at[pl.ds((c * 8 + d) * n + col0, _G * _LANES)],
                wsems[bw])
    for cp in pending_w:
        if cp is not None:
            cp.wait()


def _mf_kernel(u_id_hbm, i_id_hbm, uf_hbm, if_hbm, out_hbm,
               uid_v, iid_v, uidx_v, iidx_v, urows_v, irows_v, out_v, sem):
    wid = lax.axis_index("s") * _NC + lax.axis_index("c")
    base = wid * _BPW

    pltpu.sync_copy(u_id_hbm.at[pl.ds(base, _BPW)], uid_v)
    pltpu.sync_copy(i_id_hbm.at[pl.ds(base, _BPW)], iid_v)

    def gen(g, carry):
        uid = uid_v[pl.ds(g * _L, _L)]
        iid = iid_v[pl.ds(g * _L, _L)]
        for e in range(EMB):
            uidx_v[pl.ds(e * _BPW + g * _L, _L)] = uid + e * N_USERS
            iidx_v[pl.ds(e * _BPW + g * _L, _L)] = iid + e * N_ITEMS
        return carry

    lax.fori_loop(0, _BPW // _L, gen, 0, unroll=False)

    copies = []
    for c in range(_NXFER):
        sl = pl.ds(c * _IDX_CHUNK, _IDX_CHUNK)
        copies.append(pltpu.async_copy(uf_hbm.at[uidx_v.at[sl]],
                                       urows_v.at[sl], sem))
        copies.append(pltpu.async_copy(if_hbm.at[iidx_v.at[sl]],
                                       irows_v.at[sl], sem))
    for cp in copies:
        cp.wait()

    def body(g, carry):
        acc = jnp.zeros((_L,), jnp.float32)
        for e in range(EMB):
            sl = pl.ds(e * _BPW + g * _L, _L)
            acc = acc + urows_v[sl] * irows_v[sl]
        out_v[pl.ds(g * _L, _L)] = acc
        return carry

    lax.fori_loop(0, _BPW // _L, body, 0, unroll=False)

    pltpu.sync_copy(out_v, out_hbm.at[pl.ds(base, _BPW)])


@functools.partial(jax.jit)
def kernel(u_id, i_id, user_factors, item_factors):
    u_id = u_id.astype(jnp.int32)
    i_id = i_id.astype(jnp.int32)
    mesh = plsc.VectorSubcoreMesh(core_axis_name="c", subcore_axis_name="s")

    uT3 = user_factors.T.reshape(_NBAND, 8, N_USERS)
    iT3 = item_factors.T.reshape(_NBAND, 8, N_ITEMS)

    detile = pl.kernel(
        _detile_kernel,
        mesh=mesh,
        out_type=(jax.ShapeDtypeStruct((N_USERS * EMB,), jnp.float32),
                  jax.ShapeDtypeStruct((N_ITEMS * EMB,), jnp.float32)),
        scratch_types=(
            [pltpu.VMEM((_G * _LANES,), jnp.float32)] * _NBUF
            + [pltpu.VMEM((_TAIL,), jnp.float32)]
            + [pltpu.SemaphoreType.DMA] * (2 * _NBUF)
        ),
        compiler_params=pltpu.CompilerParams(needs_layout_passes=False),
    )
    uf_flat, if_flat = detile(uT3, iT3)

    run = pl.kernel(
        _mf_kernel,
        mesh=mesh,
        out_type=jax.ShapeDtypeStruct((BATCH,), jnp.float32),
        scratch_types=[
            pltpu.VMEM((_BPW,), jnp.int32),
            pltpu.VMEM((_BPW,), jnp.int32),
            pltpu.VMEM((_BPW * EMB,), jnp.int32),
            pltpu.VMEM((_BPW * EMB,), jnp.int32),
            pltpu.VMEM((_BPW * EMB,), jnp.float32),
            pltpu.VMEM((_BPW * EMB,), jnp.float32),
            pltpu.VMEM((_BPW,), jnp.float32),
            pltpu.SemaphoreType.DMA,
        ],
        compiler_params=pltpu.CompilerParams(
            needs_layout_passes=False, use_tc_tiling_on_sc=False),
    )
    return run(u_id, i_id, uf_flat, if_flat)
